# Initial kernel scaffold; baseline (speedup 1.0000x reference)
import jax
import jax.numpy as jnp
from jax.experimental import pallas as pl


def kernel(inputs, W0, W1, W2, W3):
    raise NotImplementedError("write your pallas kernel here")


# speedup vs baseline: 50.4309x; 50.4309x over previous
import jax
import jax.numpy as jnp
from jax import lax
from jax.experimental import pallas as pl
from jax.experimental.pallas import tpu as pltpu
from jax.experimental.pallas import tpu_sc as plsc

NC, NS, L = 2, 16, 16
NW = NC * NS
ROWS, COLS = 16384, 50
B = ROWS * COLS
H = 64
PER_W = B // NW
CHUNK = 3200
NCHUNK = PER_W // CHUNK
SUB = 128
SUB_SHIFT = 7
VPC = CHUNK // L
GSZ = 250000
CAP = CHUNK + SUB


def _body(idx_hbm, w0, w1, w2, w3, out_hbm,
          idxbuf, b0i, b0p, b1i, b1p, b2i, b2p, b3i, b3p,
          rows, gsem, ssem):
    tables = (w0, w1, w2, w3)
    bufs = ((b0i, b0p), (b1i, b1p), (b2i, b2p), (b3i, b3p))
    wid = lax.axis_index("s") * NC + lax.axis_index("c")
    iota = lax.iota(jnp.int32, L)

    def chunk_body(c, _):
        off = wid * PER_W + c * CHUNK
        pltpu.sync_copy(idx_hbm.at[pl.ds(off, CHUNK)], idxbuf)

        def compact(j, cur):
            v = idxbuf[pl.ds(j * L, L)]
            ge1 = v >= GSZ
            ge2 = v >= 2 * GSZ
            ge3 = v >= 3 * GSZ
            local = (v - jnp.where(ge1, GSZ, 0) - jnp.where(ge2, GSZ, 0)
                     - jnp.where(ge3, GSZ, 0))
            pos = off + j * L + iota
            masks = (~ge1, ge1 & ~ge2, ge2 & ~ge3, ge3)
            new = []
            for t in range(4):
                m = masks[t]
                ct = cur[t]
                ones = jnp.where(m, 1, 0)
                pref = plsc.cumsum(ones)
                dest = jnp.where(m, ct + pref - 1, CAP - 1)
                dr = lax.shift_right_logical(dest, SUB_SHIFT)
                dc = jnp.bitwise_and(dest, SUB - 1)
                plsc.store_scatter(bufs[t][0], [dr, dc], local)
                plsc.store_scatter(bufs[t][1], [dr, dc], pos)
                new.append(ct + jnp.sum(ones))
            return tuple(new)

        counts = lax.fori_loop(0, VPC, compact, (jnp.int32(0),) * 4)

        for t in range(4):
            bidx, bpos = bufs[t]
            nt = counts[t]
            full = jnp.bitwise_and(nt + (SUB - 1), jnp.int32(-SUB))
            lastsel = jnp.maximum(nt - 1, 0) + jnp.zeros((L,), jnp.int32)
            lsr = lax.shift_right_logical(lastsel, SUB_SHIFT)
            lsc = jnp.bitwise_and(lastsel, SUB - 1)
            lastv = plsc.load_gather(bidx, [lsr, lsc])
            lastp = plsc.load_gather(bpos, [lsr, lsc])
            for jj in range(SUB // L):
                lanes = nt + jj * L + iota
                lr = lax.shift_right_logical(lanes, SUB_SHIFT)
                lc = jnp.bitwise_and(lanes, SUB - 1)
                plsc.store_scatter(bidx, [lr, lc], lastv)
                plsc.store_scatter(bpos, [lr, lc], lastp)
            msub = lax.shift_right_logical(full, SUB_SHIFT)

            def sub_body(k, _):
                pltpu.async_copy(tables[t].at[bidx.at[k]], rows, gsem).wait()
                pltpu.async_copy(rows, out_hbm.at[bpos.at[k]], ssem).wait()
                return 0

            lax.fori_loop(0, msub, sub_body, 0)
        return 0

    lax.fori_loop(0, NCHUNK, chunk_body, 0)


def kernel(inputs, W0, W1, W2, W3):
    flat = inputs.reshape(-1)
    mesh = plsc.VectorSubcoreMesh(core_axis_name="c", subcore_axis_name="s")
    out = pl.kernel(
        _body,
        out_type=jax.ShapeDtypeStruct((B, H), jnp.float32),
        mesh=mesh,
        compiler_params=pltpu.CompilerParams(needs_layout_passes=False, use_tc_tiling_on_sc=False),
        scratch_types=[
            pltpu.VMEM((CHUNK,), jnp.int32),
            pltpu.VMEM((CAP // SUB, SUB), jnp.int32),
            pltpu.VMEM((CAP // SUB, SUB), jnp.int32),
            pltpu.VMEM((CAP // SUB, SUB), jnp.int32),
            pltpu.VMEM((CAP // SUB, SUB), jnp.int32),
            pltpu.VMEM((CAP // SUB, SUB), jnp.int32),
            pltpu.VMEM((CAP // SUB, SUB), jnp.int32),
            pltpu.VMEM((CAP // SUB, SUB), jnp.int32),
            pltpu.VMEM((CAP // SUB, SUB), jnp.int32),
            pltpu.VMEM((SUB, H), jnp.float32),
            pltpu.SemaphoreType.DMA,
            pltpu.SemaphoreType.DMA,
        ],
    )(flat, W0, W1, W2, W3)
    return out.reshape(ROWS, COLS, H)


# speedup vs baseline: 54.8793x; 1.0882x over previous
import jax
import jax.numpy as jnp
from jax import lax
from jax.experimental import pallas as pl
from jax.experimental.pallas import tpu as pltpu
from jax.experimental.pallas import tpu_sc as plsc

NC, NS, L = 2, 16, 16
NW = NC * NS
ROWS, COLS = 16384, 50
B = ROWS * COLS
H = 64
PER_W = B // NW
CHUNK = 3200
NCHUNK = PER_W // CHUNK
SUB = 128
SUB_SHIFT = 7
VPC = CHUNK // L
GSZ = 250000
CAP = CHUNK + SUB


def _body(idx_hbm, w0, w1, w2, w3, out_hbm,
          idxbuf, b0i, b0p, b1i, b1p, b2i, b2p, b3i, b3p,
          rows0, rows1, gsem0, gsem1, ssem0, ssem1):
    tables = (w0, w1, w2, w3)
    bufs = ((b0i, b0p), (b1i, b1p), (b2i, b2p), (b3i, b3p))
    rowbufs = (rows0, rows1)
    gsems = (gsem0, gsem1)
    ssems = (ssem0, ssem1)
    wid = lax.axis_index("s") * NC + lax.axis_index("c")
    iota = lax.iota(jnp.int32, L)

    def chunk_body(c, _):
        off = wid * PER_W + c * CHUNK
        pltpu.sync_copy(idx_hbm.at[pl.ds(off, CHUNK)], idxbuf)

        def compact(j, cur):
            v = idxbuf[pl.ds(j * L, L)]
            ge1 = v >= GSZ
            ge2 = v >= 2 * GSZ
            ge3 = v >= 3 * GSZ
            local = (v - jnp.where(ge1, GSZ, 0) - jnp.where(ge2, GSZ, 0)
                     - jnp.where(ge3, GSZ, 0))
            pos = off + j * L + iota
            masks = (~ge1, ge1 & ~ge2, ge2 & ~ge3, ge3)
            new = []
            for t in range(4):
                m = masks[t]
                ct = cur[t]
                ones = jnp.where(m, 1, 0)
                pref = plsc.cumsum(ones)
                dest = jnp.where(m, ct + pref - 1, CAP - 1)
                dr = lax.shift_right_logical(dest, SUB_SHIFT)
                dc = jnp.bitwise_and(dest, SUB - 1)
                plsc.store_scatter(bufs[t][0], [dr, dc], local)
                plsc.store_scatter(bufs[t][1], [dr, dc], pos)
                new.append(ct + jnp.sum(ones))
            return tuple(new)

        counts = lax.fori_loop(0, VPC, compact, (jnp.int32(0),) * 4)

        for t in range(4):
            bidx, bpos = bufs[t]
            nt = counts[t]
            full = jnp.bitwise_and(nt + (SUB - 1), jnp.int32(-SUB))
            lastsel = jnp.maximum(nt - 1, 0) + jnp.zeros((L,), jnp.int32)
            lsr = lax.shift_right_logical(lastsel, SUB_SHIFT)
            lsc = jnp.bitwise_and(lastsel, SUB - 1)
            lastv = plsc.load_gather(bidx, [lsr, lsc])
            lastp = plsc.load_gather(bpos, [lsr, lsc])
            for jj in range(SUB // L):
                lanes = nt + jj * L + iota
                lr = lax.shift_right_logical(lanes, SUB_SHIFT)
                lc = jnp.bitwise_and(lanes, SUB - 1)
                plsc.store_scatter(bidx, [lr, lc], lastv)
                plsc.store_scatter(bpos, [lr, lc], lastp)
            msub = lax.shift_right_logical(full, SUB_SHIFT)

            def pair_body(k2, _):
                for b in range(2):
                    k = 2 * k2 + b

                    @pl.when(k < msub)
                    def _():
                        @pl.when(k >= 2)
                        def _():
                            pltpu.make_async_copy(
                                rowbufs[b], out_hbm.at[bpos.at[k]], ssems[b]
                            ).wait()

                        pltpu.async_copy(
                            tables[t].at[bidx.at[k]], rowbufs[b], gsems[b]
                        )

                        @pl.when(k >= 1)
                        def _():
                            pltpu.make_async_copy(
                                tables[t].at[bidx.at[k]], rowbufs[1 - b],
                                gsems[1 - b],
                            ).wait()
                            pltpu.async_copy(
                                rowbufs[1 - b], out_hbm.at[bpos.at[k - 1]],
                                ssems[1 - b],
                            )
                return 0

            lax.fori_loop(0, (msub + 1) >> 1, pair_body, 0)

            lastb = jnp.bitwise_and(msub - 1, 1)
            for b in range(2):
                @pl.when(jnp.logical_and(msub >= 1, lastb == b))
                def _():
                    pltpu.make_async_copy(
                        tables[t].at[bidx.at[0]], rowbufs[b], gsems[b]
                    ).wait()
                    pltpu.async_copy(
                        rowbufs[b], out_hbm.at[bpos.at[msub - 1]], ssems[b]
                    )

            @pl.when(msub >= 2)
            def _():
                pltpu.make_async_copy(rows0, out_hbm.at[bpos.at[0]], ssems[0]).wait()
                pltpu.make_async_copy(rows1, out_hbm.at[bpos.at[0]], ssems[1]).wait()

            @pl.when(msub == 1)
            def _():
                pltpu.make_async_copy(rows0, out_hbm.at[bpos.at[0]], ssems[0]).wait()
        return 0

    lax.fori_loop(0, NCHUNK, chunk_body, 0)


def kernel(inputs, W0, W1, W2, W3):
    flat = inputs.reshape(-1)
    mesh = plsc.VectorSubcoreMesh(core_axis_name="c", subcore_axis_name="s")
    out = pl.kernel(
        _body,
        out_type=jax.ShapeDtypeStruct((B, H), jnp.float32),
        mesh=mesh,
        compiler_params=pltpu.CompilerParams(needs_layout_passes=False, use_tc_tiling_on_sc=False),
        scratch_types=[
            pltpu.VMEM((CHUNK,), jnp.int32),
            pltpu.VMEM((CAP // SUB, SUB), jnp.int32),
            pltpu.VMEM((CAP // SUB, SUB), jnp.int32),
            pltpu.VMEM((CAP // SUB, SUB), jnp.int32),
            pltpu.VMEM((CAP // SUB, SUB), jnp.int32),
            pltpu.VMEM((CAP // SUB, SUB), jnp.int32),
            pltpu.VMEM((CAP // SUB, SUB), jnp.int32),
            pltpu.VMEM((CAP // SUB, SUB), jnp.int32),
            pltpu.VMEM((CAP // SUB, SUB), jnp.int32),
            pltpu.VMEM((SUB, H), jnp.float32),
            pltpu.VMEM((SUB, H), jnp.float32),
            pltpu.SemaphoreType.DMA,
            pltpu.SemaphoreType.DMA,
            pltpu.SemaphoreType.DMA,
            pltpu.SemaphoreType.DMA,
        ],
    )(flat, W0, W1, W2, W3)
    return out.reshape(ROWS, COLS, H)


# speedup vs baseline: 55.2222x; 1.0062x over previous
import jax
import jax.numpy as jnp
from jax import lax
from jax.experimental import pallas as pl
from jax.experimental.pallas import tpu as pltpu
from jax.experimental.pallas import tpu_sc as plsc

NC, NS, L = 2, 16, 16
NW = NC * NS
ROWS, COLS = 16384, 50
B = ROWS * COLS
H = 64
PER_W = B // NW
CHUNK = 3200
NCHUNK = PER_W // CHUNK
SUB = 128
SUB_SHIFT = 7
VPC = CHUNK // L
GSZ = 250000
CAP = CHUNK + SUB


def _body(idx_hbm, w0, w1, w2, w3, out_hbm,
          idxbuf, b0i, b0p, b1i, b1p, b2i, b2p, b3i, b3p,
          rows0, rows1, rows2, rows3,
          gsem0, gsem1, gsem2, gsem3, ssem0, ssem1, ssem2, ssem3):
    tables = (w0, w1, w2, w3)
    bufs = ((b0i, b0p), (b1i, b1p), (b2i, b2p), (b3i, b3p))
    rowbufs = (rows0, rows1, rows2, rows3)
    gsems = (gsem0, gsem1, gsem2, gsem3)
    ssems = (ssem0, ssem1, ssem2, ssem3)
    wid = lax.axis_index("s") * NC + lax.axis_index("c")
    iota = lax.iota(jnp.int32, L)

    def chunk_body(c, _):
        off = wid * PER_W + c * CHUNK
        pltpu.sync_copy(idx_hbm.at[pl.ds(off, CHUNK)], idxbuf)

        def compact(j, cur):
            v = idxbuf[pl.ds(j * L, L)]
            ge1 = v >= GSZ
            ge2 = v >= 2 * GSZ
            ge3 = v >= 3 * GSZ
            local = (v - jnp.where(ge1, GSZ, 0) - jnp.where(ge2, GSZ, 0)
                     - jnp.where(ge3, GSZ, 0))
            pos = off + j * L + iota
            masks = (~ge1, ge1 & ~ge2, ge2 & ~ge3, ge3)
            new = []
            for t in range(4):
                m = masks[t]
                ct = cur[t]
                ones = jnp.where(m, 1, 0)
                pref = plsc.cumsum(ones)
                dest = jnp.where(m, ct + pref - 1, CAP - 1)
                dr = lax.shift_right_logical(dest, SUB_SHIFT)
                dc = jnp.bitwise_and(dest, SUB - 1)
                plsc.store_scatter(bufs[t][0], [dr, dc], local)
                plsc.store_scatter(bufs[t][1], [dr, dc], pos)
                new.append(ct + jnp.sum(ones))
            return tuple(new)

        counts = lax.fori_loop(0, VPC, compact, (jnp.int32(0),) * 4)

        for t in range(4):
            bidx, bpos = bufs[t]
            nt = counts[t]
            full = jnp.bitwise_and(nt + (SUB - 1), jnp.int32(-SUB))
            lastsel = jnp.maximum(nt - 1, 0) + jnp.zeros((L,), jnp.int32)
            lsr = lax.shift_right_logical(lastsel, SUB_SHIFT)
            lsc = jnp.bitwise_and(lastsel, SUB - 1)
            lastv = plsc.load_gather(bidx, [lsr, lsc])
            lastp = plsc.load_gather(bpos, [lsr, lsc])
            for jj in range(SUB // L):
                lanes = nt + jj * L + iota
                lr = lax.shift_right_logical(lanes, SUB_SHIFT)
                lc = jnp.bitwise_and(lanes, SUB - 1)
                plsc.store_scatter(bidx, [lr, lc], lastv)
                plsc.store_scatter(bpos, [lr, lc], lastp)
            msub = lax.shift_right_logical(full, SUB_SHIFT)

            def ring_body(k4, _):
                for b in range(4):
                    k = 4 * k4 + b
                    pb = (b - 1) % 4

                    @pl.when(k < msub)
                    def _():
                        @pl.when(k >= 4)
                        def _():
                            pltpu.make_async_copy(
                                rowbufs[b], out_hbm.at[bpos.at[k]], ssems[b]
                            ).wait()

                        pltpu.async_copy(
                            tables[t].at[bidx.at[k]], rowbufs[b], gsems[b]
                        )

                        @pl.when(k >= 1)
                        def _():
                            pltpu.make_async_copy(
                                tables[t].at[bidx.at[k]], rowbufs[pb],
                                gsems[pb],
                            ).wait()
                            pltpu.async_copy(
                                rowbufs[pb], out_hbm.at[bpos.at[k - 1]],
                                ssems[pb],
                            )
                return 0

            lax.fori_loop(0, (msub + 3) >> 2, ring_body, 0)

            lastp = jnp.bitwise_and(msub - 1, 3)
            for b in range(4):
                @pl.when(jnp.logical_and(msub >= 1, lastp == b))
                def _():
                    pltpu.make_async_copy(
                        tables[t].at[bidx.at[0]], rowbufs[b], gsems[b]
                    ).wait()
                    pltpu.async_copy(
                        rowbufs[b], out_hbm.at[bpos.at[msub - 1]], ssems[b]
                    )

            for b in range(4):
                @pl.when(msub > b)
                def _():
                    pltpu.make_async_copy(
                        rows0, out_hbm.at[bpos.at[0]], ssems[b]
                    ).wait()
        return 0

    lax.fori_loop(0, NCHUNK, chunk_body, 0)


def kernel(inputs, W0, W1, W2, W3):
    flat = inputs.reshape(-1)
    mesh = plsc.VectorSubcoreMesh(core_axis_name="c", subcore_axis_name="s")
    out = pl.kernel(
        _body,
        out_type=jax.ShapeDtypeStruct((B, H), jnp.float32),
        mesh=mesh,
        compiler_params=pltpu.CompilerParams(needs_layout_passes=False, use_tc_tiling_on_sc=False),
        scratch_types=[
            pltpu.VMEM((CHUNK,), jnp.int32),
            pltpu.VMEM((CAP // SUB, SUB), jnp.int32),
            pltpu.VMEM((CAP // SUB, SUB), jnp.int32),
            pltpu.VMEM((CAP // SUB, SUB), jnp.int32),
            pltpu.VMEM((CAP // SUB, SUB), jnp.int32),
            pltpu.VMEM((CAP // SUB, SUB), jnp.int32),
            pltpu.VMEM((CAP // SUB, SUB), jnp.int32),
            pltpu.VMEM((CAP // SUB, SUB), jnp.int32),
            pltpu.VMEM((CAP // SUB, SUB), jnp.int32),
            pltpu.VMEM((SUB, H), jnp.float32),
            pltpu.VMEM((SUB, H), jnp.float32),
            pltpu.VMEM((SUB, H), jnp.float32),
            pltpu.VMEM((SUB, H), jnp.float32),
            pltpu.SemaphoreType.DMA,
            pltpu.SemaphoreType.DMA,
            pltpu.SemaphoreType.DMA,
            pltpu.SemaphoreType.DMA,
            pltpu.SemaphoreType.DMA,
            pltpu.SemaphoreType.DMA,
            pltpu.SemaphoreType.DMA,
            pltpu.SemaphoreType.DMA,
        ],
    )(flat, W0, W1, W2, W3)
    return out.reshape(ROWS, COLS, H)


# speedup vs baseline: 56.3938x; 1.0212x over previous
import jax
import jax.numpy as jnp
from jax import lax
from jax.experimental import pallas as pl
from jax.experimental.pallas import tpu as pltpu
from jax.experimental.pallas import tpu_sc as plsc

NC, NS, L = 2, 16, 16
NW = NC * NS
ROWS, COLS = 16384, 50
B = ROWS * COLS
H = 64
PER_W = B // NW
CHUNK = 3200
NCHUNK = PER_W // CHUNK
SUB = 128
SUB_SHIFT = 7
VPC = CHUNK // L
GSZ = 250000
CAP = CHUNK + SUB


def _body(idx_hbm, w0, w1, w2, w3, out_hbm,
          idxbuf, b0i, b0p, b1i, b1p, b2i, b2p, b3i, b3p,
          rows0, rows1, rows2, rows3,
          gsem0, gsem1, gsem2, gsem3, ssem0, ssem1, ssem2, ssem3):
    tables = (w0, w1, w2, w3)
    bufs = ((b0i, b0p), (b1i, b1p), (b2i, b2p), (b3i, b3p))
    rowbufs = (rows0, rows1, rows2, rows3)
    gsems = (gsem0, gsem1, gsem2, gsem3)
    ssems = (ssem0, ssem1, ssem2, ssem3)
    wid = lax.axis_index("s") * NC + lax.axis_index("c")
    iota = lax.iota(jnp.int32, L)

    def chunk_body(c, _):
        off = wid * PER_W + c * CHUNK
        pltpu.sync_copy(idx_hbm.at[pl.ds(off, CHUNK)], idxbuf)

        def compact(j, cur):
            v = idxbuf[pl.ds(j * L, L)]
            ge1 = v >= GSZ
            ge2 = v >= 2 * GSZ
            ge3 = v >= 3 * GSZ
            local = (v - jnp.where(ge1, GSZ, 0) - jnp.where(ge2, GSZ, 0)
                     - jnp.where(ge3, GSZ, 0))
            pos = off + j * L + iota
            masks = (~ge1, ge1 & ~ge2, ge2 & ~ge3, ge3)
            new = []
            for t in range(4):
                m = masks[t]
                ct = cur[t]
                ones = jnp.where(m, 1, 0)
                pref = plsc.cumsum(ones)
                dest = jnp.where(m, ct + pref - 1, CAP - 1)
                dr = lax.shift_right_logical(dest, SUB_SHIFT)
                dc = jnp.bitwise_and(dest, SUB - 1)
                plsc.store_scatter(bufs[t][0], [dr, dc], local)
                plsc.store_scatter(bufs[t][1], [dr, dc], pos)
                new.append(ct + jnp.sum(ones))
            return tuple(new)

        counts = lax.fori_loop(0, VPC, compact, (jnp.int32(0),) * 4)

        msubs = []
        for t in range(4):
            bidx, bpos = bufs[t]
            nt = counts[t]
            full = jnp.bitwise_and(nt + (SUB - 1), jnp.int32(-SUB))
            lastsel = jnp.maximum(nt - 1, 0) + jnp.zeros((L,), jnp.int32)
            lsr = lax.shift_right_logical(lastsel, SUB_SHIFT)
            lsc = jnp.bitwise_and(lastsel, SUB - 1)
            lastv = plsc.load_gather(bidx, [lsr, lsc])
            lastp = plsc.load_gather(bpos, [lsr, lsc])
            for jj in range(SUB // L):
                lanes = nt + jj * L + iota
                lr = lax.shift_right_logical(lanes, SUB_SHIFT)
                lc = jnp.bitwise_and(lanes, SUB - 1)
                plsc.store_scatter(bidx, [lr, lc], lastv)
                plsc.store_scatter(bpos, [lr, lc], lastp)
            msubs.append(lax.shift_right_logical(full, SUB_SHIFT))

        cum1 = msubs[0]
        cum2 = cum1 + msubs[1]
        cum3 = cum2 + msubs[2]
        stot = cum3 + msubs[3]

        def slot_desc(i):
            d1 = i >= cum1
            d2 = i >= cum2
            d3 = i >= cum3
            t = (jnp.where(d1, 1, 0) + jnp.where(d2, 1, 0)
                 + jnp.where(d3, 1, 0))
            kk = i - jnp.where(d3, cum3,
                               jnp.where(d2, cum2, jnp.where(d1, cum1, 0)))
            return t, kk

        def ring_body(i4, carry):
            tp, kp = carry
            for b in range(4):
                i = 4 * i4 + b
                pb = (b - 1) % 4
                ti, ki = slot_desc(i)

                @pl.when(i < stot)
                def _():
                    @pl.when(i >= 4)
                    def _():
                        pltpu.make_async_copy(
                            rowbufs[b], out_hbm.at[b0p.at[0]], ssems[b]
                        ).wait()

                    for tt in range(4):
                        @pl.when(ti == tt)
                        def _():
                            pltpu.async_copy(
                                tables[tt].at[bufs[tt][0].at[ki]],
                                rowbufs[b], gsems[b],
                            )

                    @pl.when(i >= 1)
                    def _():
                        pltpu.make_async_copy(
                            tables[0].at[b0i.at[0]], rowbufs[pb], gsems[pb]
                        ).wait()
                        for tt in range(4):
                            @pl.when(tp == tt)
                            def _():
                                pltpu.async_copy(
                                    rowbufs[pb],
                                    out_hbm.at[bufs[tt][1].at[kp]],
                                    ssems[pb],
                                )

                tp, kp = (jnp.where(i < stot, ti, tp),
                          jnp.where(i < stot, ki, kp))
            return tp, kp

        tlast, klast = lax.fori_loop(
            0, (stot + 3) >> 2, ring_body, (jnp.int32(0), jnp.int32(0))
        )

        lastp4 = jnp.bitwise_and(stot - 1, 3)
        for b in range(4):
            @pl.when(lastp4 == b)
            def _():
                pltpu.make_async_copy(
                    tables[0].at[b0i.at[0]], rowbufs[b], gsems[b]
                ).wait()
                for tt in range(4):
                    @pl.when(tlast == tt)
                    def _():
                        pltpu.async_copy(
                            rowbufs[b], out_hbm.at[bufs[tt][1].at[klast]],
                            ssems[b],
                        )

        for b in range(4):
            @pl.when(stot > b)
            def _():
                pltpu.make_async_copy(
                    rows0, out_hbm.at[b0p.at[0]], ssems[b]
                ).wait()
        return 0

    lax.fori_loop(0, NCHUNK, chunk_body, 0)


def kernel(inputs, W0, W1, W2, W3):
    flat = inputs.reshape(-1)
    mesh = plsc.VectorSubcoreMesh(core_axis_name="c", subcore_axis_name="s")
    out = pl.kernel(
        _body,
        out_type=jax.ShapeDtypeStruct((B, H), jnp.float32),
        mesh=mesh,
        compiler_params=pltpu.CompilerParams(
            needs_layout_passes=False, use_tc_tiling_on_sc=False),
        scratch_types=[
            pltpu.VMEM((CHUNK,), jnp.int32),
            pltpu.VMEM((CAP // SUB, SUB), jnp.int32),
            pltpu.VMEM((CAP // SUB, SUB), jnp.int32),
            pltpu.VMEM((CAP // SUB, SUB), jnp.int32),
            pltpu.VMEM((CAP // SUB, SUB), jnp.int32),
            pltpu.VMEM((CAP // SUB, SUB), jnp.int32),
            pltpu.VMEM((CAP // SUB, SUB), jnp.int32),
            pltpu.VMEM((CAP // SUB, SUB), jnp.int32),
            pltpu.VMEM((CAP // SUB, SUB), jnp.int32),
            pltpu.VMEM((SUB, H), jnp.float32),
            pltpu.VMEM((SUB, H), jnp.float32),
            pltpu.VMEM((SUB, H), jnp.float32),
            pltpu.VMEM((SUB, H), jnp.float32),
            pltpu.SemaphoreType.DMA,
            pltpu.SemaphoreType.DMA,
            pltpu.SemaphoreType.DMA,
            pltpu.SemaphoreType.DMA,
            pltpu.SemaphoreType.DMA,
            pltpu.SemaphoreType.DMA,
            pltpu.SemaphoreType.DMA,
            pltpu.SemaphoreType.DMA,
        ],
    )(flat, W0, W1, W2, W3)
    return out.reshape(ROWS, COLS, H)


# speedup vs baseline: 61.6356x; 1.0930x over previous
import jax
import jax.numpy as jnp
from jax import lax
from jax.experimental import pallas as pl
from jax.experimental.pallas import tpu as pltpu
from jax.experimental.pallas import tpu_sc as plsc

NC, NS, L = 2, 16, 16
NW = NC * NS
ROWS, COLS = 16384, 50
B = ROWS * COLS
H = 64
PER_W = B // NW
CHUNK = 6400
NCHUNK = PER_W // CHUNK
SUB = 128
SUB_SHIFT = 7
VPC = CHUNK // L
GSZ = 250000
CAP = CHUNK + SUB


def _body(idx_hbm, w0, w1, w2, w3, out_hbm,
          idxbuf, b0i, b0p, b1i, b1p, b2i, b2p, b3i, b3p,
          rows0, rows1, rows2, rows3,
          gsem0, gsem1, gsem2, gsem3, ssem0, ssem1, ssem2, ssem3):
    tables = (w0, w1, w2, w3)
    bufs = ((b0i, b0p), (b1i, b1p), (b2i, b2p), (b3i, b3p))
    rowbufs = (rows0, rows1, rows2, rows3)
    gsems = (gsem0, gsem1, gsem2, gsem3)
    ssems = (ssem0, ssem1, ssem2, ssem3)
    wid = lax.axis_index("s") * NC + lax.axis_index("c")
    iota = lax.iota(jnp.int32, L)

    def chunk_body(c, _):
        off = wid * PER_W + c * CHUNK
        pltpu.sync_copy(idx_hbm.at[pl.ds(off, CHUNK)], idxbuf)

        def compact(j, cur):
            v = idxbuf[pl.ds(j * L, L)]
            ge1 = v >= GSZ
            ge2 = v >= 2 * GSZ
            ge3 = v >= 3 * GSZ
            local = (v - jnp.where(ge1, GSZ, 0) - jnp.where(ge2, GSZ, 0)
                     - jnp.where(ge3, GSZ, 0))
            pos = off + j * L + iota
            masks = (~ge1, ge1 & ~ge2, ge2 & ~ge3, ge3)
            new = []
            for t in range(4):
                m = masks[t]
                ct = cur[t]
                ones = jnp.where(m, 1, 0)
                pref = plsc.cumsum(ones)
                dest = jnp.where(m, ct + pref - 1, CAP - 1)
                dr = lax.shift_right_logical(dest, SUB_SHIFT)
                dc = jnp.bitwise_and(dest, SUB - 1)
                plsc.store_scatter(bufs[t][0], [dr, dc], local)
                plsc.store_scatter(bufs[t][1], [dr, dc], pos)
                new.append(ct + jnp.sum(ones))
            return tuple(new)

        counts = lax.fori_loop(0, VPC, compact, (jnp.int32(0),) * 4)

        msubs = []
        for t in range(4):
            bidx, bpos = bufs[t]
            nt = counts[t]
            full = jnp.bitwise_and(nt + (SUB - 1), jnp.int32(-SUB))
            lastsel = jnp.maximum(nt - 1, 0) + jnp.zeros((L,), jnp.int32)
            lsr = lax.shift_right_logical(lastsel, SUB_SHIFT)
            lsc = jnp.bitwise_and(lastsel, SUB - 1)
            lastv = plsc.load_gather(bidx, [lsr, lsc])
            lastp = plsc.load_gather(bpos, [lsr, lsc])
            for jj in range(SUB // L):
                lanes = nt + jj * L + iota
                lr = lax.shift_right_logical(lanes, SUB_SHIFT)
                lc = jnp.bitwise_and(lanes, SUB - 1)
                plsc.store_scatter(bidx, [lr, lc], lastv)
                plsc.store_scatter(bpos, [lr, lc], lastp)
            msubs.append(lax.shift_right_logical(full, SUB_SHIFT))

        cum1 = msubs[0]
        cum2 = cum1 + msubs[1]
        cum3 = cum2 + msubs[2]
        stot = cum3 + msubs[3]

        def slot_desc(i):
            d1 = i >= cum1
            d2 = i >= cum2
            d3 = i >= cum3
            t = (jnp.where(d1, 1, 0) + jnp.where(d2, 1, 0)
                 + jnp.where(d3, 1, 0))
            kk = i - jnp.where(d3, cum3,
                               jnp.where(d2, cum2, jnp.where(d1, cum1, 0)))
            return t, kk

        def ring_body(i4, carry):
            tp, kp = carry
            for b in range(4):
                i = 4 * i4 + b
                pb = (b - 1) % 4
                ti, ki = slot_desc(i)

                @pl.when(i < stot)
                def _():
                    @pl.when(i >= 4)
                    def _():
                        pltpu.make_async_copy(
                            rowbufs[b], out_hbm.at[b0p.at[0]], ssems[b]
                        ).wait()

                    for tt in range(4):
                        @pl.when(ti == tt)
                        def _():
                            pltpu.async_copy(
                                tables[tt].at[bufs[tt][0].at[ki]],
                                rowbufs[b], gsems[b],
                            )

                    @pl.when(i >= 1)
                    def _():
                        pltpu.make_async_copy(
                            tables[0].at[b0i.at[0]], rowbufs[pb], gsems[pb]
                        ).wait()
                        for tt in range(4):
                            @pl.when(tp == tt)
                            def _():
                                pltpu.async_copy(
                                    rowbufs[pb],
                                    out_hbm.at[bufs[tt][1].at[kp]],
                                    ssems[pb],
                                )

                tp, kp = (jnp.where(i < stot, ti, tp),
                          jnp.where(i < stot, ki, kp))
            return tp, kp

        tlast, klast = lax.fori_loop(
            0, (stot + 3) >> 2, ring_body, (jnp.int32(0), jnp.int32(0))
        )

        lastp4 = jnp.bitwise_and(stot - 1, 3)
        for b in range(4):
            @pl.when(lastp4 == b)
            def _():
                pltpu.make_async_copy(
                    tables[0].at[b0i.at[0]], rowbufs[b], gsems[b]
                ).wait()
                for tt in range(4):
                    @pl.when(tlast == tt)
                    def _():
                        pltpu.async_copy(
                            rowbufs[b], out_hbm.at[bufs[tt][1].at[klast]],
                            ssems[b],
                        )

        for b in range(4):
            @pl.when(stot > b)
            def _():
                pltpu.make_async_copy(
                    rows0, out_hbm.at[b0p.at[0]], ssems[b]
                ).wait()
        return 0

    lax.fori_loop(0, NCHUNK, chunk_body, 0)


def kernel(inputs, W0, W1, W2, W3):
    flat = inputs.reshape(-1)
    mesh = plsc.VectorSubcoreMesh(core_axis_name="c", subcore_axis_name="s")
    out = pl.kernel(
        _body,
        out_type=jax.ShapeDtypeStruct((B, H), jnp.float32),
        mesh=mesh,
        compiler_params=pltpu.CompilerParams(
            needs_layout_passes=False, use_tc_tiling_on_sc=False),
        scratch_types=[
            pltpu.VMEM((CHUNK,), jnp.int32),
            pltpu.VMEM((CAP // SUB, SUB), jnp.int32),
            pltpu.VMEM((CAP // SUB, SUB), jnp.int32),
            pltpu.VMEM((CAP // SUB, SUB), jnp.int32),
            pltpu.VMEM((CAP // SUB, SUB), jnp.int32),
            pltpu.VMEM((CAP // SUB, SUB), jnp.int32),
            pltpu.VMEM((CAP // SUB, SUB), jnp.int32),
            pltpu.VMEM((CAP // SUB, SUB), jnp.int32),
            pltpu.VMEM((CAP // SUB, SUB), jnp.int32),
            pltpu.VMEM((SUB, H), jnp.float32),
            pltpu.VMEM((SUB, H), jnp.float32),
            pltpu.VMEM((SUB, H), jnp.float32),
            pltpu.VMEM((SUB, H), jnp.float32),
            pltpu.SemaphoreType.DMA,
            pltpu.SemaphoreType.DMA,
            pltpu.SemaphoreType.DMA,
            pltpu.SemaphoreType.DMA,
            pltpu.SemaphoreType.DMA,
            pltpu.SemaphoreType.DMA,
            pltpu.SemaphoreType.DMA,
            pltpu.SemaphoreType.DMA,
        ],
    )(flat, W0, W1, W2, W3)
    return out.reshape(ROWS, COLS, H)


# speedup vs baseline: 61.9468x; 1.0050x over previous
import jax
import jax.numpy as jnp
from jax import lax
from jax.experimental import pallas as pl
from jax.experimental.pallas import tpu as pltpu
from jax.experimental.pallas import tpu_sc as plsc

NC, NS, L = 2, 16, 16
NW = NC * NS
ROWS, COLS = 16384, 50
B = ROWS * COLS
H = 64
PER_W = B // NW
CHUNK = 6400
NCHUNK = PER_W // CHUNK
SUB = 128
SUB_SHIFT = 7
VPC = CHUNK // L
GSZ = 250000
CAP = CHUNK + SUB
NSR = CAP // SUB


def _body(idx_hbm, w0, w1, w2, w3, out_hbm,
          idxbuf, bigidx, bigpos,
          rows0, rows1, rows2, rows3,
          gsem0, gsem1, gsem2, gsem3, ssem0, ssem1, ssem2, ssem3):
    tables = (w0, w1, w2, w3)
    rowbufs = (rows0, rows1, rows2, rows3)
    gsems = (gsem0, gsem1, gsem2, gsem3)
    ssems = (ssem0, ssem1, ssem2, ssem3)
    wid = lax.axis_index("s") * NC + lax.axis_index("c")
    iota = lax.iota(jnp.int32, L)

    def chunk_body(c, _):
        off = wid * PER_W + c * CHUNK
        pltpu.sync_copy(idx_hbm.at[pl.ds(off, CHUNK)], idxbuf)

        def compact(j, cur):
            v = idxbuf[pl.ds(j * L, L)]
            g = (jnp.where(v >= GSZ, 1, 0) + jnp.where(v >= 2 * GSZ, 1, 0)
                 + jnp.where(v >= 3 * GSZ, 1, 0))
            local = v - g * GSZ
            pos = off + j * L + iota
            g8 = g * 8
            w = lax.shift_left(jnp.full((L,), 1, jnp.int32), g8)
            pref = plsc.cumsum(w)
            prefsel = jnp.bitwise_and(lax.shift_right_logical(pref, g8), 255)
            ctsel = jnp.where(g == 0, cur[0],
                              jnp.where(g == 1, cur[1],
                                        jnp.where(g == 2, cur[2], cur[3])))
            dest = g * CAP + ctsel + prefsel - 1
            dr = lax.shift_right_logical(dest, SUB_SHIFT)
            dc = jnp.bitwise_and(dest, SUB - 1)
            plsc.store_scatter(bigidx, [dr, dc], local)
            plsc.store_scatter(bigpos, [dr, dc], pos)
            tot = jnp.sum(w)
            return tuple(
                cur[t] + jnp.bitwise_and(
                    lax.shift_right_logical(tot, 8 * t), 255)
                for t in range(4))

        counts = lax.fori_loop(0, VPC, compact, (jnp.int32(0),) * 4)

        msubs = []
        for t in range(4):
            nt = counts[t]
            base = t * CAP
            full = jnp.bitwise_and(nt + (SUB - 1), jnp.int32(-SUB))
            lastsel = base + jnp.maximum(nt - 1, 0) + jnp.zeros((L,), jnp.int32)
            lsr = lax.shift_right_logical(lastsel, SUB_SHIFT)
            lsc = jnp.bitwise_and(lastsel, SUB - 1)
            lastv = plsc.load_gather(bigidx, [lsr, lsc])
            lastp = plsc.load_gather(bigpos, [lsr, lsc])
            for jj in range(SUB // L):
                lanes = base + nt + jj * L + iota
                lr = lax.shift_right_logical(lanes, SUB_SHIFT)
                lc = jnp.bitwise_and(lanes, SUB - 1)
                plsc.store_scatter(bigidx, [lr, lc], lastv)
                plsc.store_scatter(bigpos, [lr, lc], lastp)
            msubs.append(lax.shift_right_logical(full, SUB_SHIFT))

        cum1 = msubs[0]
        cum2 = cum1 + msubs[1]
        cum3 = cum2 + msubs[2]
        stot = cum3 + msubs[3]

        def slot_desc(i):
            d1 = i >= cum1
            d2 = i >= cum2
            d3 = i >= cum3
            t = (jnp.where(d1, 1, 0) + jnp.where(d2, 1, 0)
                 + jnp.where(d3, 1, 0))
            kk = i - jnp.where(d3, cum3,
                               jnp.where(d2, cum2, jnp.where(d1, cum1, 0)))
            return t, kk

        def ring_body(i4, carry):
            tp, kp = carry
            for b in range(4):
                i = 4 * i4 + b
                pb = (b - 1) % 4
                ti, ki = slot_desc(i)

                @pl.when(i < stot)
                def _():
                    @pl.when(i >= 4)
                    def _():
                        pltpu.make_async_copy(
                            rowbufs[b], out_hbm.at[bigpos.at[0]], ssems[b]
                        ).wait()

                    gi = ti * NSR + ki
                    for tt in range(4):
                        @pl.when(ti == tt)
                        def _():
                            pltpu.async_copy(
                                tables[tt].at[bigidx.at[gi]],
                                rowbufs[b], gsems[b],
                            )

                    @pl.when(i >= 1)
                    def _():
                        pltpu.make_async_copy(
                            tables[0].at[bigidx.at[0]], rowbufs[pb], gsems[pb]
                        ).wait()
                        pltpu.async_copy(
                            rowbufs[pb],
                            out_hbm.at[bigpos.at[tp * NSR + kp]],
                            ssems[pb],
                        )

                tp, kp = (jnp.where(i < stot, ti, tp),
                          jnp.where(i < stot, ki, kp))
            return tp, kp

        tlast, klast = lax.fori_loop(
            0, (stot + 3) >> 2, ring_body, (jnp.int32(0), jnp.int32(0))
        )

        lastp4 = jnp.bitwise_and(stot - 1, 3)
        for b in range(4):
            @pl.when(lastp4 == b)
            def _():
                pltpu.make_async_copy(
                    tables[0].at[bigidx.at[0]], rowbufs[b], gsems[b]
                ).wait()
                pltpu.async_copy(
                    rowbufs[b], out_hbm.at[bigpos.at[tlast * NSR + klast]],
                    ssems[b],
                )

        for b in range(4):
            @pl.when(stot > b)
            def _():
                pltpu.make_async_copy(
                    rows0, out_hbm.at[bigpos.at[0]], ssems[b]
                ).wait()
        return 0

    lax.fori_loop(0, NCHUNK, chunk_body, 0)


def kernel(inputs, W0, W1, W2, W3):
    flat = inputs.reshape(-1)
    mesh = plsc.VectorSubcoreMesh(core_axis_name="c", subcore_axis_name="s")
    out = pl.kernel(
        _body,
        out_type=jax.ShapeDtypeStruct((B, H), jnp.float32),
        mesh=mesh,
        compiler_params=pltpu.CompilerParams(
            needs_layout_passes=False, use_tc_tiling_on_sc=False),
        scratch_types=[
            pltpu.VMEM((CHUNK,), jnp.int32),
            pltpu.VMEM((4 * NSR, SUB), jnp.int32),
            pltpu.VMEM((4 * NSR, SUB), jnp.int32),
            pltpu.VMEM((SUB, H), jnp.float32),
            pltpu.VMEM((SUB, H), jnp.float32),
            pltpu.VMEM((SUB, H), jnp.float32),
            pltpu.VMEM((SUB, H), jnp.float32),
            pltpu.SemaphoreType.DMA,
            pltpu.SemaphoreType.DMA,
            pltpu.SemaphoreType.DMA,
            pltpu.SemaphoreType.DMA,
            pltpu.SemaphoreType.DMA,
            pltpu.SemaphoreType.DMA,
            pltpu.SemaphoreType.DMA,
            pltpu.SemaphoreType.DMA,
        ],
    )(flat, W0, W1, W2, W3)
    return out.reshape(ROWS, COLS, H)


# speedup vs baseline: 63.6754x; 1.0279x over previous
import jax
import jax.numpy as jnp
from jax import lax
from jax.experimental import pallas as pl
from jax.experimental.pallas import tpu as pltpu
from jax.experimental.pallas import tpu_sc as plsc

NC, NS, L = 2, 16, 16
NW = NC * NS
ROWS, COLS = 16384, 50
B = ROWS * COLS
H = 64
PER_W = B // NW
CHUNK = 6400
NCHUNK = PER_W // CHUNK
SUB = 128
SUB_SHIFT = 7
VPC = CHUNK // L
GSZ = 250000
CAP = CHUNK + SUB
NSR = CAP // SUB


def _body(idx_hbm, w0, w1, w2, w3, out_hbm,
          idxbuf, bigidx, bigpos,
          rows0, rows1, rows2, rows3,
          gsem0, gsem1, gsem2, gsem3, ssem0, ssem1, ssem2, ssem3):
    tables = (w0, w1, w2, w3)
    rowbufs = (rows0, rows1, rows2, rows3)
    gsems = (gsem0, gsem1, gsem2, gsem3)
    ssems = (ssem0, ssem1, ssem2, ssem3)
    wid = lax.axis_index("s") * NC + lax.axis_index("c")
    iota = lax.iota(jnp.int32, L)

    def chunk_body(c, _):
        off = wid * PER_W + c * CHUNK
        pltpu.sync_copy(idx_hbm.at[pl.ds(off, CHUNK)], idxbuf)

        def compact(j, cur):
            v = idxbuf[pl.ds(j * L, L)]
            g = (jnp.where(v >= GSZ, 1, 0) + jnp.where(v >= 2 * GSZ, 1, 0)
                 + jnp.where(v >= 3 * GSZ, 1, 0))
            local = v - g * GSZ
            pos = off + j * L + iota
            g8 = g * 8
            w = lax.shift_left(jnp.full((L,), 1, jnp.int32), g8)
            pref = plsc.cumsum(w)
            prefsel = jnp.bitwise_and(lax.shift_right_logical(pref, g8), 255)
            ctsel = jnp.where(g == 0, cur[0],
                              jnp.where(g == 1, cur[1],
                                        jnp.where(g == 2, cur[2], cur[3])))
            dest = g * CAP + ctsel + prefsel - 1
            dr = lax.shift_right_logical(dest, SUB_SHIFT)
            dc = jnp.bitwise_and(dest, SUB - 1)
            plsc.store_scatter(bigidx, [dr, dc], local)
            plsc.store_scatter(bigpos, [dr, dc], pos)
            tot = jnp.sum(w)
            return tuple(
                cur[t] + jnp.bitwise_and(
                    lax.shift_right_logical(tot, 8 * t), 255)
                for t in range(4))

        counts = lax.fori_loop(0, VPC, compact, (jnp.int32(0),) * 4)

        msubs = []
        for t in range(4):
            nt = counts[t]
            base = t * CAP
            full = jnp.bitwise_and(nt + (SUB - 1), jnp.int32(-SUB))
            lastsel = base + jnp.maximum(nt - 1, 0) + jnp.zeros((L,), jnp.int32)
            lsr = lax.shift_right_logical(lastsel, SUB_SHIFT)
            lsc = jnp.bitwise_and(lastsel, SUB - 1)
            lastv = plsc.load_gather(bigidx, [lsr, lsc])
            lastp = plsc.load_gather(bigpos, [lsr, lsc])
            for jj in range(SUB // L):
                lanes = base + nt + jj * L + iota
                lr = lax.shift_right_logical(lanes, SUB_SHIFT)
                lc = jnp.bitwise_and(lanes, SUB - 1)
                plsc.store_scatter(bigidx, [lr, lc], lastv)
                plsc.store_scatter(bigpos, [lr, lc], lastp)
            msubs.append(lax.shift_right_logical(full, SUB_SHIFT))

        cum1 = msubs[0]
        cum2 = cum1 + msubs[1]
        cum3 = cum2 + msubs[2]
        stot = cum3 + msubs[3]

        def slot_desc(i):
            d1 = i >= cum1
            d2 = i >= cum2
            d3 = i >= cum3
            t = (jnp.where(d1, 1, 0) + jnp.where(d2, 1, 0)
                 + jnp.where(d3, 1, 0))
            kk = i - jnp.where(d3, cum3,
                               jnp.where(d2, cum2, jnp.where(d1, cum1, 0)))
            return t, kk

        def ring_body(i4, carry):
            t1, k1, t2, k2 = carry
            for b in range(4):
                i = 4 * i4 + b
                pb2 = (b - 2) % 4
                ti, ki = slot_desc(i)

                @pl.when(i < stot)
                def _():
                    @pl.when(i >= 4)
                    def _():
                        pltpu.make_async_copy(
                            rowbufs[b], out_hbm.at[bigpos.at[0]], ssems[b]
                        ).wait()

                    gi = ti * NSR + ki
                    for tt in range(4):
                        @pl.when(ti == tt)
                        def _():
                            pltpu.async_copy(
                                tables[tt].at[bigidx.at[gi]],
                                rowbufs[b], gsems[b],
                            )

                    @pl.when(i >= 2)
                    def _():
                        pltpu.make_async_copy(
                            tables[0].at[bigidx.at[0]], rowbufs[pb2],
                            gsems[pb2],
                        ).wait()
                        pltpu.async_copy(
                            rowbufs[pb2],
                            out_hbm.at[bigpos.at[t2 * NSR + k2]],
                            ssems[pb2],
                        )

                live = i < stot
                t2, k2 = (jnp.where(live, t1, t2), jnp.where(live, k1, k2))
                t1, k1 = (jnp.where(live, ti, t1), jnp.where(live, ki, k1))
            return t1, k1, t2, k2

        t1, k1, t2, k2 = lax.fori_loop(
            0, (stot + 3) >> 2, ring_body,
            (jnp.int32(0),) * 4,
        )

        p1 = jnp.bitwise_and(stot - 1, 3)
        p2 = jnp.bitwise_and(stot - 2, 3)
        for b in range(4):
            @pl.when(jnp.logical_and(stot >= 2, p2 == b))
            def _():
                pltpu.make_async_copy(
                    tables[0].at[bigidx.at[0]], rowbufs[b], gsems[b]
                ).wait()
                pltpu.async_copy(
                    rowbufs[b], out_hbm.at[bigpos.at[t2 * NSR + k2]],
                    ssems[b],
                )
        for b in range(4):
            @pl.when(p1 == b)
            def _():
                pltpu.make_async_copy(
                    tables[0].at[bigidx.at[0]], rowbufs[b], gsems[b]
                ).wait()
                pltpu.async_copy(
                    rowbufs[b], out_hbm.at[bigpos.at[t1 * NSR + k1]],
                    ssems[b],
                )

        for b in range(4):
            @pl.when(stot > b)
            def _():
                pltpu.make_async_copy(
                    rows0, out_hbm.at[bigpos.at[0]], ssems[b]
                ).wait()
        return 0

    lax.fori_loop(0, NCHUNK, chunk_body, 0)


def kernel(inputs, W0, W1, W2, W3):
    flat = inputs.reshape(-1)
    mesh = plsc.VectorSubcoreMesh(core_axis_name="c", subcore_axis_name="s")
    out = pl.kernel(
        _body,
        out_type=jax.ShapeDtypeStruct((B, H), jnp.float32),
        mesh=mesh,
        compiler_params=pltpu.CompilerParams(
            needs_layout_passes=False, use_tc_tiling_on_sc=False),
        scratch_types=[
            pltpu.VMEM((CHUNK,), jnp.int32),
            pltpu.VMEM((4 * NSR, SUB), jnp.int32),
            pltpu.VMEM((4 * NSR, SUB), jnp.int32),
            pltpu.VMEM((SUB, H), jnp.float32),
            pltpu.VMEM((SUB, H), jnp.float32),
            pltpu.VMEM((SUB, H), jnp.float32),
            pltpu.VMEM((SUB, H), jnp.float32),
            pltpu.SemaphoreType.DMA,
            pltpu.SemaphoreType.DMA,
            pltpu.SemaphoreType.DMA,
            pltpu.SemaphoreType.DMA,
            pltpu.SemaphoreType.DMA,
            pltpu.SemaphoreType.DMA,
            pltpu.SemaphoreType.DMA,
            pltpu.SemaphoreType.DMA,
        ],
    )(flat, W0, W1, W2, W3)
    return out.reshape(ROWS, COLS, H)


# speedup vs baseline: 64.2999x; 1.0098x over previous
import jax
import jax.numpy as jnp
from jax import lax
from jax.experimental import pallas as pl
from jax.experimental.pallas import tpu as pltpu
from jax.experimental.pallas import tpu_sc as plsc

NC, NS, L = 2, 16, 16
NW = NC * NS
ROWS, COLS = 16384, 50
B = ROWS * COLS
H = 64
PER_W = B // NW
CHUNK = 6400
NCHUNK = PER_W // CHUNK
SUB = 128
SUB_SHIFT = 7
VPC = CHUNK // L
GSZ = 250000
CAP = CHUNK + SUB
NSR = CAP // SUB


def _body(idx_hbm, w0, w1, w2, w3, out_hbm,
          idxbuf, bigidx, bigpos,
          rows0, rows1, rows2, rows3,
          gsem0, gsem1, gsem2, gsem3, ssem0, ssem1, ssem2, ssem3):
    tables = (w0, w1, w2, w3)
    rowbufs = (rows0, rows1, rows2, rows3)
    gsems = (gsem0, gsem1, gsem2, gsem3)
    ssems = (ssem0, ssem1, ssem2, ssem3)
    wid = lax.axis_index("s") * NC + lax.axis_index("c")
    iota = lax.iota(jnp.int32, L)

    def chunk_body(c, _):
        off = wid * PER_W + c * CHUNK
        pltpu.sync_copy(idx_hbm.at[pl.ds(off, CHUNK)], idxbuf)

        def compact(j, cur):
            v = idxbuf[pl.ds(j * L, L)]
            g = (jnp.where(v >= GSZ, 1, 0) + jnp.where(v >= 2 * GSZ, 1, 0)
                 + jnp.where(v >= 3 * GSZ, 1, 0))
            local = v - g * GSZ
            pos = off + j * L + iota
            g8 = g * 8
            w = lax.shift_left(jnp.full((L,), 1, jnp.int32), g8)
            pref = plsc.cumsum(w)
            prefsel = jnp.bitwise_and(lax.shift_right_logical(pref, g8), 255)
            ctsel = jnp.where(g == 0, cur[0],
                              jnp.where(g == 1, cur[1],
                                        jnp.where(g == 2, cur[2], cur[3])))
            dest = g * CAP + ctsel + prefsel - 1
            dr = lax.shift_right_logical(dest, SUB_SHIFT)
            dc = jnp.bitwise_and(dest, SUB - 1)
            plsc.store_scatter(bigidx, [dr, dc], local)
            plsc.store_scatter(bigpos, [dr, dc], pos)
            tot = jnp.sum(w)
            return tuple(
                cur[t] + jnp.bitwise_and(
                    lax.shift_right_logical(tot, 8 * t), 255)
                for t in range(4))

        counts = lax.fori_loop(0, VPC, compact, (jnp.int32(0),) * 4)

        msubs = []
        for t in range(4):
            nt = counts[t]
            base = t * CAP
            full = jnp.bitwise_and(nt + (SUB - 1), jnp.int32(-SUB))
            lastsel = base + jnp.maximum(nt - 1, 0) + jnp.zeros((L,), jnp.int32)
            lsr = lax.shift_right_logical(lastsel, SUB_SHIFT)
            lsc = jnp.bitwise_and(lastsel, SUB - 1)
            lastv = plsc.load_gather(bigidx, [lsr, lsc])
            lastp = plsc.load_gather(bigpos, [lsr, lsc])
            for jj in range(SUB // L):
                lanes = base + nt + jj * L + iota
                lr = lax.shift_right_logical(lanes, SUB_SHIFT)
                lc = jnp.bitwise_and(lanes, SUB - 1)
                plsc.store_scatter(bigidx, [lr, lc], lastv)
                plsc.store_scatter(bigpos, [lr, lc], lastp)
            msubs.append(lax.shift_right_logical(full, SUB_SHIFT))

        cum1 = msubs[0]
        cum2 = cum1 + msubs[1]
        cum3 = cum2 + msubs[2]
        stot = cum3 + msubs[3]

        def slot_desc(i):
            d1 = i >= cum1
            d2 = i >= cum2
            d3 = i >= cum3
            t = (jnp.where(d1, 1, 0) + jnp.where(d2, 1, 0)
                 + jnp.where(d3, 1, 0))
            kk = i - jnp.where(d3, cum3,
                               jnp.where(d2, cum2, jnp.where(d1, cum1, 0)))
            return t, kk

        def ring_body(i4, carry):
            t1, k1, t2, k2, t3, k3 = carry
            for b in range(4):
                i = 4 * i4 + b
                pb3 = (b - 3) % 4
                ti, ki = slot_desc(i)

                @pl.when(i < stot)
                def _():
                    @pl.when(i >= 4)
                    def _():
                        pltpu.make_async_copy(
                            rowbufs[b], out_hbm.at[bigpos.at[0]], ssems[b]
                        ).wait()

                    gi = ti * NSR + ki
                    for tt in range(4):
                        @pl.when(ti == tt)
                        def _():
                            pltpu.async_copy(
                                tables[tt].at[bigidx.at[gi]],
                                rowbufs[b], gsems[b],
                            )

                    @pl.when(i >= 3)
                    def _():
                        pltpu.make_async_copy(
                            tables[0].at[bigidx.at[0]], rowbufs[pb3],
                            gsems[pb3],
                        ).wait()
                        pltpu.async_copy(
                            rowbufs[pb3],
                            out_hbm.at[bigpos.at[t3 * NSR + k3]],
                            ssems[pb3],
                        )

                live = i < stot
                t3, k3 = (jnp.where(live, t2, t3), jnp.where(live, k2, k3))
                t2, k2 = (jnp.where(live, t1, t2), jnp.where(live, k1, k2))
                t1, k1 = (jnp.where(live, ti, t1), jnp.where(live, ki, k1))
            return t1, k1, t2, k2, t3, k3

        t1, k1, t2, k2, t3, k3 = lax.fori_loop(
            0, (stot + 3) >> 2, ring_body,
            (jnp.int32(0),) * 6,
        )

        for d, (td, kd) in ((3, (t3, k3)), (2, (t2, k2)), (1, (t1, k1))):
            pd = jnp.bitwise_and(stot - d, 3)
            for b in range(4):
                @pl.when(jnp.logical_and(stot >= d, pd == b))
                def _():
                    pltpu.make_async_copy(
                        tables[0].at[bigidx.at[0]], rowbufs[b], gsems[b]
                    ).wait()
                    pltpu.async_copy(
                        rowbufs[b], out_hbm.at[bigpos.at[td * NSR + kd]],
                        ssems[b],
                    )

        for b in range(4):
            @pl.when(stot > b)
            def _():
                pltpu.make_async_copy(
                    rows0, out_hbm.at[bigpos.at[0]], ssems[b]
                ).wait()
        return 0

    lax.fori_loop(0, NCHUNK, chunk_body, 0)


def kernel(inputs, W0, W1, W2, W3):
    flat = inputs.reshape(-1)
    mesh = plsc.VectorSubcoreMesh(core_axis_name="c", subcore_axis_name="s")
    out = pl.kernel(
        _body,
        out_type=jax.ShapeDtypeStruct((B, H), jnp.float32),
        mesh=mesh,
        compiler_params=pltpu.CompilerParams(
            needs_layout_passes=False, use_tc_tiling_on_sc=False),
        scratch_types=[
            pltpu.VMEM((CHUNK,), jnp.int32),
            pltpu.VMEM((4 * NSR, SUB), jnp.int32),
            pltpu.VMEM((4 * NSR, SUB), jnp.int32),
            pltpu.VMEM((SUB, H), jnp.float32),
            pltpu.VMEM((SUB, H), jnp.float32),
            pltpu.VMEM((SUB, H), jnp.float32),
            pltpu.VMEM((SUB, H), jnp.float32),
            pltpu.SemaphoreType.DMA,
            pltpu.SemaphoreType.DMA,
            pltpu.SemaphoreType.DMA,
            pltpu.SemaphoreType.DMA,
            pltpu.SemaphoreType.DMA,
            pltpu.SemaphoreType.DMA,
            pltpu.SemaphoreType.DMA,
            pltpu.SemaphoreType.DMA,
        ],
    )(flat, W0, W1, W2, W3)
    return out.reshape(ROWS, COLS, H)
